# baseline (speedup 1.0000x reference)
import functools

import jax
import jax.numpy as jnp
from jax import lax
from jax.experimental import pallas as pl
from jax.experimental.pallas import tpu as pltpu
from jax.experimental.pallas import tpu_sc as plsc

B, D, H, W = 8, 256, 32, 32
K = 8192
N = B * H * W
COMMITMENT_COST = 0.25

TN = 256
NT1 = N // TN
TN3 = 512
NT3 = N // TN3

SC_CORES = 2
SC_SUBCORES = 16
SC_WORKERS = SC_CORES * SC_SUBCORES
ROWS_PER_WORKER = N // SC_WORKERS


def _k1_body(x2_ref, e2_ref, x_ref, w_ref, dist_ref, idx_ref, enc_ref,
             cnt_ref, cnt_acc):
    n = pl.program_id(0)
    m = lax.dot_general(
        x_ref[...], w_ref[...],
        dimension_numbers=(((1,), (1,)), ((), ())),
        preferred_element_type=jnp.float32)
    t = x2_ref[...] + e2_ref[...]
    dist = t - 2.0 * m
    dist_ref[...] = dist
    rowmin = jnp.min(dist, axis=1, keepdims=True)
    col = lax.broadcasted_iota(jnp.int32, (TN, K), 1)
    cand = jnp.where(dist == rowmin, col, jnp.int32(K))
    arg = jnp.min(cand, axis=1, keepdims=True)
    idx_ref[...] = arg
    onehot = (col == arg).astype(jnp.float32)
    enc_ref[...] = onehot
    part = jnp.sum(onehot, axis=0, keepdims=True)

    @pl.when(n == 0)
    def _():
        cnt_acc[...] = part

    @pl.when(n > 0)
    def _():
        cnt_acc[...] = cnt_acc[...] + part

    @pl.when(n == NT1 - 1)
    def _():
        cnt_ref[...] = cnt_acc[...]


def _run_k1(x2, e2, flat, w):
    return pl.pallas_call(
        _k1_body,
        grid=(NT1,),
        in_specs=[
            pl.BlockSpec((TN, 1), lambda n: (n, 0)),
            pl.BlockSpec((1, K), lambda n: (0, 0)),
            pl.BlockSpec((TN, D), lambda n: (n, 0)),
            pl.BlockSpec((K, D), lambda n: (0, 0)),
        ],
        out_specs=[
            pl.BlockSpec((TN, K), lambda n: (n, 0)),
            pl.BlockSpec((TN, 1), lambda n: (n, 0)),
            pl.BlockSpec((TN, K), lambda n: (n, 0)),
            pl.BlockSpec((1, K), lambda n: (0, 0)),
        ],
        out_shape=[
            jax.ShapeDtypeStruct((N, K), jnp.float32),
            jax.ShapeDtypeStruct((N, 1), jnp.int32),
            jax.ShapeDtypeStruct((N, K), jnp.float32),
            jax.ShapeDtypeStruct((1, K), jnp.float32),
        ],
        scratch_shapes=[pltpu.VMEM((1, K), jnp.float32)],
    )(x2, e2, flat, w)


def _sc_gather(table, idx2d):
    mesh = plsc.VectorSubcoreMesh(core_axis_name="c", subcore_axis_name="s")

    @functools.partial(
        pl.kernel,
        mesh=mesh,
        out_type=jax.ShapeDtypeStruct((N, D), jnp.float32),
        scratch_types=[
            pltpu.VMEM((128,), jnp.int32),
            pltpu.VMEM((128,), jnp.int32),
            pltpu.VMEM((ROWS_PER_WORKER, D), jnp.float32),
            pltpu.SemaphoreType.DMA,
        ],
    )
    def g(table_hbm, idx_hbm, out_hbm, idx_va, idx_vb, rows_v, sem):
        wid = lax.axis_index("s") * SC_CORES + lax.axis_index("c")
        pltpu.sync_copy(idx_hbm.at[2 * wid], idx_va)
        pltpu.sync_copy(idx_hbm.at[2 * wid + 1], idx_vb)
        c1 = pltpu.async_copy(table_hbm.at[idx_va], rows_v.at[pl.ds(0, 128)], sem)
        c2 = pltpu.async_copy(table_hbm.at[idx_vb], rows_v.at[pl.ds(128, 128)], sem)
        c1.wait()
        c2.wait()
        pltpu.sync_copy(rows_v, out_hbm.at[pl.ds(wid * ROWS_PER_WORKER,
                                                 ROWS_PER_WORKER)])

    return g(table, idx2d)


DBLK = 128
HW = H * W
NT3R = B * (D // DBLK)


def _k3_body(x_ref, q_ref, cnt_ref, qst_ref, loss_ref, perp_ref, acc):
    i = pl.program_id(0)
    x = x_ref[0]
    qt = jnp.transpose(q_ref[...], (1, 0))
    d = qt - x
    qst_ref[0] = x + d
    s = jnp.sum(d * d)

    @pl.when(i == 0)
    def _():
        acc[0] = s

    @pl.when(i > 0)
    def _():
        acc[0] = acc[0] + s

    @pl.when(i == NT3R - 1)
    def _():
        lat = acc[0] * (1.0 / (N * D))
        loss_ref[0, 0] = lat + COMMITMENT_COST * lat
        p = cnt_ref[...] * (1.0 / N)
        ent = jnp.sum(p * jnp.log(p + 1e-10))
        perp_ref[0, 0] = jnp.exp(-ent)


def _run_k3(x_raw, q, counts):
    nd = D // DBLK
    return pl.pallas_call(
        _k3_body,
        grid=(NT3R,),
        in_specs=[
            pl.BlockSpec((1, DBLK, HW), lambda i: (i // nd, i % nd, 0)),
            pl.BlockSpec((HW, DBLK), lambda i: (i // nd, i % nd)),
            pl.BlockSpec((1, K), lambda i: (0, 0)),
        ],
        out_specs=[
            pl.BlockSpec((1, DBLK, HW), lambda i: (i // nd, i % nd, 0)),
            pl.BlockSpec(memory_space=pltpu.SMEM),
            pl.BlockSpec(memory_space=pltpu.SMEM),
        ],
        out_shape=[
            jax.ShapeDtypeStruct((B, D, HW), jnp.float32),
            jax.ShapeDtypeStruct((1, 1), jnp.float32),
            jax.ShapeDtypeStruct((1, 1), jnp.float32),
        ],
        scratch_shapes=[pltpu.SMEM((1,), jnp.float32)],
    )(x_raw, q, counts)


def kernel(inputs, embedding_weight):
    x = jnp.transpose(inputs, (0, 2, 3, 1))
    flat = x.reshape(-1, D)
    x2 = jnp.sum(flat ** 2, axis=1, keepdims=True)
    e2 = jnp.sum(embedding_weight ** 2, axis=1).reshape(1, K)

    distances, idx, encodings, counts = _run_k1(x2, e2, flat, embedding_weight)

    q = _sc_gather(embedding_weight, idx.reshape(64, 128))

    qst_raw, loss11, perp11 = _run_k3(inputs.reshape(B, D, HW), q, counts)

    quantized_st = qst_raw.reshape(B, D, H, W)
    return (loss11.reshape(()),
            quantized_st,
            perp11.reshape(()),
            distances,
            idx,
            encodings)


# speedup vs baseline: 1.0839x; 1.0839x over previous
import functools

import jax
import jax.numpy as jnp
from jax import lax
from jax.experimental import pallas as pl
from jax.experimental.pallas import tpu as pltpu
from jax.experimental.pallas import tpu_sc as plsc

B, D, H, W = 8, 256, 32, 32
K = 8192
N = B * H * W
COMMITMENT_COST = 0.25

TN = 256
NT1 = N // TN
TN3 = 512
NT3 = N // TN3

SC_CORES = 2
SC_SUBCORES = 16
SC_WORKERS = SC_CORES * SC_SUBCORES
ROWS_PER_WORKER = N // SC_WORKERS


def _k1_body(x2_ref, e2_ref, x_ref, w_ref, dist_ref, idx_ref, enc_ref,
             cnt_ref, cnt_acc):
    n = pl.program_id(0)
    m = lax.dot_general(
        x_ref[...], w_ref[...],
        dimension_numbers=(((1,), (1,)), ((), ())),
        preferred_element_type=jnp.float32)
    t = x2_ref[...] + e2_ref[...]
    dist = t - 2.0 * m
    dist_ref[...] = dist
    rowmin = jnp.min(dist, axis=1, keepdims=True)
    col = lax.broadcasted_iota(jnp.int32, (TN, K), 1)
    cand = jnp.where(dist == rowmin, col, jnp.int32(K))
    arg = jnp.min(cand, axis=1, keepdims=True)
    idx_ref[...] = arg
    onehot = (col == arg).astype(jnp.float32)
    enc_ref[...] = onehot
    part = jnp.sum(onehot, axis=0, keepdims=True)

    @pl.when(n == 0)
    def _():
        cnt_acc[...] = part

    @pl.when(n > 0)
    def _():
        cnt_acc[...] = cnt_acc[...] + part

    @pl.when(n == NT1 - 1)
    def _():
        cnt_ref[...] = cnt_acc[...]


def _run_k1(x2, e2, flat, w):
    return pl.pallas_call(
        _k1_body,
        grid=(NT1,),
        in_specs=[
            pl.BlockSpec((TN, 1), lambda n: (n, 0)),
            pl.BlockSpec((1, K), lambda n: (0, 0)),
            pl.BlockSpec((TN, D), lambda n: (n, 0)),
            pl.BlockSpec((K, D), lambda n: (0, 0)),
        ],
        out_specs=[
            pl.BlockSpec((TN, K), lambda n: (n, 0)),
            pl.BlockSpec((TN, 1), lambda n: (n, 0)),
            pl.BlockSpec((TN, K), lambda n: (n, 0)),
            pl.BlockSpec((1, K), lambda n: (0, 0)),
        ],
        out_shape=[
            jax.ShapeDtypeStruct((N, K), jnp.float32),
            jax.ShapeDtypeStruct((N, 1), jnp.int32),
            jax.ShapeDtypeStruct((N, K), jnp.float32),
            jax.ShapeDtypeStruct((1, K), jnp.float32),
        ],
        scratch_shapes=[pltpu.VMEM((1, K), jnp.float32)],
    )(x2, e2, flat, w)


def _sc_gather(table, idx2d):
    mesh = plsc.VectorSubcoreMesh(core_axis_name="c", subcore_axis_name="s")

    @functools.partial(
        pl.kernel,
        mesh=mesh,
        out_type=jax.ShapeDtypeStruct((N, D), jnp.float32),
        scratch_types=[
            pltpu.VMEM((128,), jnp.int32),
            pltpu.VMEM((128,), jnp.int32),
            pltpu.VMEM((ROWS_PER_WORKER, D), jnp.float32),
            pltpu.SemaphoreType.DMA,
        ],
    )
    def g(table_hbm, idx_hbm, out_hbm, idx_va, idx_vb, rows_v, sem):
        wid = lax.axis_index("s") * SC_CORES + lax.axis_index("c")
        pltpu.sync_copy(idx_hbm.at[2 * wid], idx_va)
        pltpu.sync_copy(idx_hbm.at[2 * wid + 1], idx_vb)
        c1 = pltpu.async_copy(table_hbm.at[idx_va], rows_v.at[pl.ds(0, 128)], sem)
        c2 = pltpu.async_copy(table_hbm.at[idx_vb], rows_v.at[pl.ds(128, 128)], sem)
        c1.wait()
        c2.wait()
        pltpu.sync_copy(rows_v, out_hbm.at[pl.ds(wid * ROWS_PER_WORKER,
                                                 ROWS_PER_WORKER)])

    return g(table, idx2d)


def _k3_body(x_ref, q_ref, cnt_ref, loss_ref, perp_ref, acc):
    i = pl.program_id(0)
    d = q_ref[...] - x_ref[...]
    s = jnp.sum(d * d)

    @pl.when(i == 0)
    def _():
        acc[0] = s

    @pl.when(i > 0)
    def _():
        acc[0] = acc[0] + s

    @pl.when(i == NT3 - 1)
    def _():
        lat = acc[0] * (1.0 / (N * D))
        loss_ref[0, 0] = lat + COMMITMENT_COST * lat
        p = cnt_ref[...] * (1.0 / N)
        ent = jnp.sum(p * jnp.log(p + 1e-10))
        perp_ref[0, 0] = jnp.exp(-ent)


def _run_k3(flat, q, counts):
    return pl.pallas_call(
        _k3_body,
        grid=(NT3,),
        in_specs=[
            pl.BlockSpec((TN3, D), lambda i: (i, 0)),
            pl.BlockSpec((TN3, D), lambda i: (i, 0)),
            pl.BlockSpec((1, K), lambda i: (0, 0)),
        ],
        out_specs=[
            pl.BlockSpec(memory_space=pltpu.SMEM),
            pl.BlockSpec(memory_space=pltpu.SMEM),
        ],
        out_shape=[
            jax.ShapeDtypeStruct((1, 1), jnp.float32),
            jax.ShapeDtypeStruct((1, 1), jnp.float32),
        ],
        scratch_shapes=[pltpu.SMEM((1,), jnp.float32)],
    )(flat, q, counts)


def kernel(inputs, embedding_weight):
    x = jnp.transpose(inputs, (0, 2, 3, 1))
    flat = x.reshape(-1, D)
    x2 = jnp.sum(flat ** 2, axis=1, keepdims=True)
    e2 = jnp.sum(embedding_weight ** 2, axis=1).reshape(1, K)

    distances, idx, encodings, counts = _run_k1(x2, e2, flat, embedding_weight)

    q = _sc_gather(embedding_weight, idx.reshape(64, 128))

    loss11, perp11 = _run_k3(flat, q, counts)

    quantized_st = jnp.transpose(q.reshape(B, H, W, D), (0, 3, 1, 2))
    return (loss11.reshape(()),
            quantized_st,
            perp11.reshape(()),
            distances,
            idx,
            encodings)


# speedup vs baseline: 1.1518x; 1.0627x over previous
import functools

import jax
import jax.numpy as jnp
from jax import lax
from jax.experimental import pallas as pl
from jax.experimental.pallas import tpu as pltpu
from jax.experimental.pallas import tpu_sc as plsc

B, D, H, W = 8, 256, 32, 32
K = 8192
N = B * H * W
COMMITMENT_COST = 0.25

TN = 256
NT1 = N // TN
TN3 = 512
NT3 = N // TN3

SC_CORES = 2
SC_SUBCORES = 16
SC_WORKERS = SC_CORES * SC_SUBCORES
ROWS_PER_WORKER = N // SC_WORKERS


def _k1_body(x2_ref, e2_ref, x_ref, w_ref, dist_ref, idx_ref, enc_ref,
             loss_ref, perp_ref, cnt_acc, loss_acc):
    n = pl.program_id(0)
    m = lax.dot_general(
        x_ref[...], w_ref[...],
        dimension_numbers=(((1,), (1,)), ((), ())),
        preferred_element_type=jnp.float32)
    t = x2_ref[...] + e2_ref[...]
    dist = t - 2.0 * m
    dist_ref[...] = dist
    rowmin = jnp.min(dist, axis=1, keepdims=True)
    col = lax.broadcasted_iota(jnp.int32, (TN, K), 1)
    cand = jnp.where(dist == rowmin, col, jnp.int32(K))
    arg = jnp.min(cand, axis=1, keepdims=True)
    idx_ref[...] = arg
    onehot = (col == arg).astype(jnp.float32)
    enc_ref[...] = onehot
    part = jnp.sum(onehot, axis=0, keepdims=True)
    rowsum = jnp.sum(rowmin)

    @pl.when(n == 0)
    def _():
        cnt_acc[...] = part
        loss_acc[0] = rowsum

    @pl.when(n > 0)
    def _():
        cnt_acc[...] = cnt_acc[...] + part
        loss_acc[0] = loss_acc[0] + rowsum

    @pl.when(n == NT1 - 1)
    def _():
        lat = loss_acc[0] * (1.0 / (N * D))
        loss_ref[0, 0] = lat + COMMITMENT_COST * lat
        p = cnt_acc[...] * (1.0 / N)
        ent = jnp.sum(p * jnp.log(p + 1e-10))
        perp_ref[0, 0] = jnp.exp(-ent)


def _run_k1(x2, e2, flat, w):
    return pl.pallas_call(
        _k1_body,
        grid=(NT1,),
        in_specs=[
            pl.BlockSpec((TN, 1), lambda n: (n, 0)),
            pl.BlockSpec((1, K), lambda n: (0, 0)),
            pl.BlockSpec((TN, D), lambda n: (n, 0)),
            pl.BlockSpec((K, D), lambda n: (0, 0)),
        ],
        out_specs=[
            pl.BlockSpec((TN, K), lambda n: (n, 0)),
            pl.BlockSpec((TN, 1), lambda n: (n, 0)),
            pl.BlockSpec((TN, K), lambda n: (n, 0)),
            pl.BlockSpec(memory_space=pltpu.SMEM),
            pl.BlockSpec(memory_space=pltpu.SMEM),
        ],
        out_shape=[
            jax.ShapeDtypeStruct((N, K), jnp.float32),
            jax.ShapeDtypeStruct((N, 1), jnp.int32),
            jax.ShapeDtypeStruct((N, K), jnp.float32),
            jax.ShapeDtypeStruct((1, 1), jnp.float32),
            jax.ShapeDtypeStruct((1, 1), jnp.float32),
        ],
        scratch_shapes=[pltpu.VMEM((1, K), jnp.float32),
                        pltpu.SMEM((1,), jnp.float32)],
    )(x2, e2, flat, w)


def _sc_gather(table, idx2d):
    mesh = plsc.VectorSubcoreMesh(core_axis_name="c", subcore_axis_name="s")

    @functools.partial(
        pl.kernel,
        mesh=mesh,
        out_type=jax.ShapeDtypeStruct((N, D), jnp.float32),
        scratch_types=[
            pltpu.VMEM((128,), jnp.int32),
            pltpu.VMEM((128,), jnp.int32),
            pltpu.VMEM((ROWS_PER_WORKER, D), jnp.float32),
            pltpu.SemaphoreType.DMA,
        ],
    )
    def g(table_hbm, idx_hbm, out_hbm, idx_va, idx_vb, rows_v, sem):
        wid = lax.axis_index("s") * SC_CORES + lax.axis_index("c")
        pltpu.sync_copy(idx_hbm.at[2 * wid], idx_va)
        pltpu.sync_copy(idx_hbm.at[2 * wid + 1], idx_vb)
        c1 = pltpu.async_copy(table_hbm.at[idx_va], rows_v.at[pl.ds(0, 128)], sem)
        c2 = pltpu.async_copy(table_hbm.at[idx_vb], rows_v.at[pl.ds(128, 128)], sem)
        c1.wait()
        c2.wait()
        pltpu.sync_copy(rows_v, out_hbm.at[pl.ds(wid * ROWS_PER_WORKER,
                                                 ROWS_PER_WORKER)])

    return g(table, idx2d)


def kernel(inputs, embedding_weight):
    x = jnp.transpose(inputs, (0, 2, 3, 1))
    flat = x.reshape(-1, D)
    x2 = jnp.sum(flat ** 2, axis=1, keepdims=True)
    e2 = jnp.sum(embedding_weight ** 2, axis=1).reshape(1, K)

    distances, idx, encodings, loss11, perp11 = _run_k1(
        x2, e2, flat, embedding_weight)

    q = _sc_gather(embedding_weight, idx.reshape(64, 128))

    quantized_st = jnp.transpose(q.reshape(B, H, W, D), (0, 3, 1, 2))
    return (loss11.reshape(()),
            quantized_st,
            perp11.reshape(()),
            distances,
            idx,
            encodings)
